# Initial kernel scaffold; baseline (speedup 1.0000x reference)
import jax
import jax.numpy as jnp
from jax.experimental import pallas as pl


def kernel(input_, weight):
    raise NotImplementedError("write your pallas kernel here")


# speedup vs baseline: 2.9652x; 2.9652x over previous
import functools

import jax
import jax.numpy as jnp
from jax import lax
from jax.experimental import pallas as pl
from jax.experimental.pallas import tpu as pltpu
from jax.experimental.pallas import tpu_sc as plsc

NUM_EMBEDDINGS = 100000
EMBEDDING_DIM = 128
BATCH = 4096
HIST = 50

_INFO = plsc.get_sparse_core_info()
_NC = _INFO.num_cores
_NS = _INFO.num_subcores
_NW = _NC * _NS
_TOTAL = BATCH * HIST
_ROWS_PER_W = _TOTAL // _NW
_CHUNK = 128
_NCHUNK = _ROWS_PER_W // _CHUNK


def _sc_gather(idx_hbm, table_hbm):
    mesh = plsc.VectorSubcoreMesh(core_axis_name="c", subcore_axis_name="s")

    @functools.partial(
        pl.kernel,
        mesh=mesh,
        out_type=jax.ShapeDtypeStruct((_TOTAL, EMBEDDING_DIM), jnp.float32),
        scratch_types=[
            pltpu.VMEM((_NCHUNK, _CHUNK), jnp.int32),
            pltpu.VMEM((_CHUNK, EMBEDDING_DIM), jnp.float32),
            pltpu.SemaphoreType.DMA,
        ],
    )
    def k(idx_ref, table_ref, out_ref, idx_v, rows_v, sem):
        wid = lax.axis_index("s") * _NC + lax.axis_index("c")
        pltpu.sync_copy(idx_ref.at[wid], idx_v)
        base = wid * _ROWS_PER_W

        def step(j, carry):
            pltpu.async_copy(table_ref.at[idx_v.at[j]], rows_v, sem).wait()
            pltpu.sync_copy(rows_v, out_ref.at[pl.ds(base + j * _CHUNK, _CHUNK)])
            return carry

        lax.fori_loop(0, _NCHUNK, step, 0)

    return k(idx_hbm, table_hbm)


def kernel(input_, weight):
    idx = input_.astype(jnp.int32).reshape(_NW, _NCHUNK, _CHUNK)
    flat = _sc_gather(idx, weight)
    return flat.reshape(BATCH, HIST, EMBEDDING_DIM)


# speedup vs baseline: 3.3519x; 1.1304x over previous
import functools

import jax
import jax.numpy as jnp
from jax import lax
from jax.experimental import pallas as pl
from jax.experimental.pallas import tpu as pltpu
from jax.experimental.pallas import tpu_sc as plsc

NUM_EMBEDDINGS = 100000
EMBEDDING_DIM = 128
BATCH = 4096
HIST = 50

_INFO = plsc.get_sparse_core_info()
_NC = _INFO.num_cores
_NS = _INFO.num_subcores
_NW = _NC * _NS
_TOTAL = BATCH * HIST
_ROWS_PER_W = _TOTAL // _NW
_CHUNK = 128
_NCHUNK = _ROWS_PER_W // _CHUNK
_NB = 6
_G = 3


def _sc_gather(idx_hbm, table_hbm):
    mesh = plsc.VectorSubcoreMesh(core_axis_name="c", subcore_axis_name="s")

    @functools.partial(
        pl.kernel,
        mesh=mesh,
        out_type=jax.ShapeDtypeStruct((_TOTAL, EMBEDDING_DIM), jnp.float32),
        scratch_types=[
            pltpu.VMEM((_NCHUNK, _CHUNK), jnp.int32),
            pltpu.VMEM((_NB, _CHUNK, EMBEDDING_DIM), jnp.float32),
            pltpu.SemaphoreType.DMA((_NB,)),
            pltpu.SemaphoreType.DMA((_NB,)),
        ],
    )
    def k(idx_ref, table_ref, out_ref, idx_v, rows_v, gsem, ssem):
        wid = lax.axis_index("s") * _NC + lax.axis_index("c")
        pltpu.sync_copy(idx_ref.at[wid], idx_v)
        base = wid * _ROWS_PER_W

        def step(j, carry):
            @pl.when(j < _NCHUNK)
            def _fire():
                b = j % _NB

                @pl.when(j >= _NB)
                def _reuse():
                    pltpu.make_async_copy(
                        rows_v.at[b], out_ref.at[pl.ds(base, _CHUNK)], ssem.at[b]
                    ).wait()

                pltpu.async_copy(table_ref.at[idx_v.at[j]], rows_v.at[b], gsem.at[b])

            @pl.when(j >= _G)
            def _drain():
                c = j - _G
                b = c % _NB
                pltpu.make_async_copy(
                    table_ref.at[idx_v.at[c]], rows_v.at[b], gsem.at[b]
                ).wait()
                pltpu.async_copy(
                    rows_v.at[b], out_ref.at[pl.ds(base + c * _CHUNK, _CHUNK)], ssem.at[b]
                )

            return carry

        lax.fori_loop(0, _NCHUNK + _G, step, 0)

        for b in range(_NB):
            pltpu.make_async_copy(
                rows_v.at[b], out_ref.at[pl.ds(base, _CHUNK)], ssem.at[b]
            ).wait()

    return k(idx_hbm, table_hbm)


def kernel(input_, weight):
    idx = input_.astype(jnp.int32).reshape(_NW, _NCHUNK, _CHUNK)
    flat = _sc_gather(idx, weight)
    return flat.reshape(BATCH, HIST, EMBEDDING_DIM)


# speedup vs baseline: 5.9561x; 1.7769x over previous
import functools

import jax
import jax.numpy as jnp
from jax import lax
from jax.experimental import pallas as pl
from jax.experimental.pallas import tpu as pltpu
from jax.experimental.pallas import tpu_sc as plsc

NUM_EMBEDDINGS = 100000
EMBEDDING_DIM = 128
BATCH = 4096
HIST = 50

_INFO = plsc.get_sparse_core_info()
_NC = _INFO.num_cores
_NS = _INFO.num_subcores
_NW = _NC * _NS
_B_PER_W = BATCH // _NW
_NB = 8
_G = 4


def _sc_gather(idx_hbm, table_hbm):
    mesh = plsc.VectorSubcoreMesh(core_axis_name="c", subcore_axis_name="s")

    @functools.partial(
        pl.kernel,
        mesh=mesh,
        out_type=jax.ShapeDtypeStruct((BATCH, HIST, EMBEDDING_DIM), jnp.float32),
        scratch_types=[
            pltpu.VMEM((_B_PER_W, HIST), jnp.int32),
            pltpu.VMEM((_NB, HIST, EMBEDDING_DIM), jnp.float32),
            pltpu.SemaphoreType.DMA((_NB,)),
            pltpu.SemaphoreType.DMA((_NB,)),
        ],
    )
    def k(idx_ref, table_ref, out_ref, idx_v, rows_v, gsem, ssem):
        wid = lax.axis_index("s") * _NC + lax.axis_index("c")
        base = wid * _B_PER_W
        pltpu.sync_copy(idx_ref.at[pl.ds(base, _B_PER_W)], idx_v)

        def step(j, carry):
            @pl.when(j < _B_PER_W)
            def _fire():
                b = j % _NB

                @pl.when(j >= _NB)
                def _reuse():
                    pltpu.make_async_copy(
                        rows_v.at[b], out_ref.at[base], ssem.at[b]
                    ).wait()

                pltpu.async_copy(table_ref.at[idx_v.at[j]], rows_v.at[b], gsem.at[b])

            @pl.when(j >= _G)
            def _drain():
                c = j - _G
                b = c % _NB
                pltpu.make_async_copy(
                    table_ref.at[idx_v.at[c]], rows_v.at[b], gsem.at[b]
                ).wait()
                pltpu.async_copy(rows_v.at[b], out_ref.at[base + c], ssem.at[b])

            return carry

        lax.fori_loop(0, _B_PER_W + _G, step, 0)

        for b in range(_NB):
            pltpu.make_async_copy(
                rows_v.at[b], out_ref.at[base], ssem.at[b]
            ).wait()

    return k(idx_hbm, table_hbm)


def kernel(input_, weight):
    return _sc_gather(input_.astype(jnp.int32), weight)


# speedup vs baseline: 5.9585x; 1.0004x over previous
import functools

import jax
import jax.numpy as jnp
from jax import lax
from jax.experimental import pallas as pl
from jax.experimental.pallas import tpu as pltpu
from jax.experimental.pallas import tpu_sc as plsc

NUM_EMBEDDINGS = 100000
EMBEDDING_DIM = 128
BATCH = 4096
HIST = 50

_INFO = plsc.get_sparse_core_info()
_NC = _INFO.num_cores
_NS = _INFO.num_subcores
_NW = _NC * _NS
_B_PER_W = BATCH // _NW
_NB = 8
_G = 4


def _sc_gather(idx_hbm, table_hbm):
    mesh = plsc.VectorSubcoreMesh(core_axis_name="c", subcore_axis_name="s")

    @functools.partial(
        pl.kernel,
        mesh=mesh,
        out_type=jax.ShapeDtypeStruct((BATCH, HIST, EMBEDDING_DIM), jnp.float32),
        compiler_params=pltpu.CompilerParams(use_tc_tiling_on_sc=True),
        scratch_types=[
            pltpu.VMEM((_B_PER_W, HIST), jnp.int32),
            pltpu.VMEM((_NB, HIST, EMBEDDING_DIM), jnp.float32),
            pltpu.SemaphoreType.DMA((_NB,)),
            pltpu.SemaphoreType.DMA((_NB,)),
        ],
    )
    def k(idx_ref, table_ref, out_ref, idx_v, rows_v, gsem, ssem):
        wid = lax.axis_index("s") * _NC + lax.axis_index("c")
        base = wid * _B_PER_W
        pltpu.sync_copy(idx_ref.at[pl.ds(base, _B_PER_W)], idx_v)

        def step(j, carry):
            @pl.when(j < _B_PER_W)
            def _fire():
                b = j % _NB

                @pl.when(j >= _NB)
                def _reuse():
                    pltpu.make_async_copy(
                        rows_v.at[b], out_ref.at[base], ssem.at[b]
                    ).wait()

                pltpu.async_copy(table_ref.at[idx_v.at[j]], rows_v.at[b], gsem.at[b])

            @pl.when(j >= _G)
            def _drain():
                c = j - _G
                b = c % _NB
                pltpu.make_async_copy(
                    table_ref.at[idx_v.at[c]], rows_v.at[b], gsem.at[b]
                ).wait()
                pltpu.async_copy(rows_v.at[b], out_ref.at[base + c], ssem.at[b])

            return carry

        lax.fori_loop(0, _B_PER_W + _G, step, 0)

        for b in range(_NB):
            pltpu.make_async_copy(
                rows_v.at[b], out_ref.at[base], ssem.at[b]
            ).wait()

    return k(idx_hbm, table_hbm)


def kernel(input_, weight):
    return _sc_gather(input_.astype(jnp.int32), weight)


# speedup vs baseline: 10.7138x; 1.7981x over previous
import functools

import jax
import jax.numpy as jnp
from jax import lax
from jax.experimental import pallas as pl
from jax.experimental.pallas import tpu as pltpu
from jax.experimental.pallas import tpu_sc as plsc

NUM_EMBEDDINGS = 100000
EMBEDDING_DIM = 128
BATCH = 4096
HIST = 50

_INFO = plsc.get_sparse_core_info()
_NC = _INFO.num_cores
_NS = _INFO.num_subcores
_NW = _NC * _NS
_B_PER_W = BATCH // _NW
_NB = 6
_G = 3


def _sc_gather(idx_hbm, table_hbm):
    mesh = plsc.VectorSubcoreMesh(core_axis_name="c", subcore_axis_name="s")

    @functools.partial(
        pl.kernel,
        mesh=mesh,
        out_type=jax.ShapeDtypeStruct((HIST, BATCH, EMBEDDING_DIM), jnp.float32),
        scratch_types=[
            pltpu.VMEM((HIST, _B_PER_W), jnp.int32),
            pltpu.VMEM((_NB, _B_PER_W, EMBEDDING_DIM), jnp.float32),
            pltpu.SemaphoreType.DMA((_NB,)),
            pltpu.SemaphoreType.DMA((_NB,)),
        ],
    )
    def k(idx_ref, table_ref, out_ref, idx_v, rows_v, gsem, ssem):
        wid = lax.axis_index("s") * _NC + lax.axis_index("c")
        base = wid * _B_PER_W
        pltpu.sync_copy(idx_ref.at[:, pl.ds(base, _B_PER_W)], idx_v)

        def step(j, carry):
            @pl.when(j < HIST)
            def _fire():
                b = j % _NB

                @pl.when(j >= _NB)
                def _reuse():
                    pltpu.make_async_copy(
                        rows_v.at[b], out_ref.at[0, pl.ds(base, _B_PER_W)], ssem.at[b]
                    ).wait()

                pltpu.async_copy(table_ref.at[idx_v.at[j]], rows_v.at[b], gsem.at[b])

            @pl.when(j >= _G)
            def _drain():
                c = j - _G
                b = c % _NB
                pltpu.make_async_copy(
                    table_ref.at[idx_v.at[c]], rows_v.at[b], gsem.at[b]
                ).wait()
                pltpu.async_copy(
                    rows_v.at[b], out_ref.at[c, pl.ds(base, _B_PER_W)], ssem.at[b]
                )

            return carry

        lax.fori_loop(0, HIST + _G, step, 0)

        for b in range(_NB):
            pltpu.make_async_copy(
                rows_v.at[b], out_ref.at[0, pl.ds(base, _B_PER_W)], ssem.at[b]
            ).wait()

    return k(idx_hbm, table_hbm)


def kernel(input_, weight):
    idx_t = input_.astype(jnp.int32).T
    out_hm = _sc_gather(idx_t, weight)
    return jnp.transpose(out_hm, (1, 0, 2))
